# Initial kernel scaffold; baseline (speedup 1.0000x reference)
import jax
import jax.numpy as jnp
from jax.experimental import pallas as pl


def kernel(pos, A):
    raise NotImplementedError("write your pallas kernel here")


# speedup vs baseline: 1.9685x; 1.9685x over previous
import functools

import jax
import jax.numpy as jnp
from jax import lax
from jax.experimental import pallas as pl
from jax.experimental.pallas import tpu as pltpu
from jax.experimental.pallas import tpu_sc as plsc

EMBED_DIM = 1024
NUM_CORES = 2
NUM_SUBCORES = 16
NW = NUM_CORES * NUM_SUBCORES
CHUNK = 32


def _sc_gather(table, idx_flat):
    B = idx_flat.shape[0]
    b_per_w = B // NW
    n_chunks = b_per_w // CHUNK
    mesh = plsc.VectorSubcoreMesh(core_axis_name="c", subcore_axis_name="s")

    @functools.partial(
        pl.kernel,
        out_type=jax.ShapeDtypeStruct((B, EMBED_DIM), jnp.float32),
        mesh=mesh,
        scratch_types=[
            pltpu.VMEM((b_per_w,), jnp.int32),
            pltpu.VMEM((2, CHUNK, EMBED_DIM), jnp.float32),
            pltpu.SemaphoreType.DMA,
            pltpu.SemaphoreType.DMA,
        ],
    )
    def k(table_hbm, idx_hbm, out_hbm, idx_v, rows_v, gsem, osem):
        wid = lax.axis_index("s") * NUM_CORES + lax.axis_index("c")
        base = wid * b_per_w
        pltpu.sync_copy(idx_hbm.at[pl.ds(base, b_per_w)], idx_v)

        out_copies = [None, None]
        for j in range(n_chunks):
            buf = j % 2
            if out_copies[buf] is not None:
                out_copies[buf].wait()
            gather = pltpu.async_copy(
                table_hbm.at[idx_v.at[pl.ds(j * CHUNK, CHUNK)]],
                rows_v.at[buf],
                gsem,
            )
            gather.wait()
            out_copies[buf] = pltpu.async_copy(
                rows_v.at[buf],
                out_hbm.at[pl.ds(base + j * CHUNK, CHUNK)],
                osem,
            )
        for c in out_copies:
            if c is not None:
                c.wait()

    return k(table, idx_flat)


def kernel(pos, A):
    batch, seq = pos.shape
    idx_flat = pos.reshape(batch * seq).astype(jnp.int32)
    out = _sc_gather(A, idx_flat)
    return out.reshape(batch, seq, EMBED_DIM)


# speedup vs baseline: 2.0624x; 1.0477x over previous
import functools

import jax
import jax.numpy as jnp
from jax import lax
from jax.experimental import pallas as pl
from jax.experimental.pallas import tpu as pltpu
from jax.experimental.pallas import tpu_sc as plsc

EMBED_DIM = 1024
NUM_CORES = 2
NUM_SUBCORES = 16
NW = NUM_CORES * NUM_SUBCORES
CHUNK = 32
NBUF = 3


def _sc_gather(table, idx_flat):
    B = idx_flat.shape[0]
    b_per_w = B // NW
    n_chunks = b_per_w // CHUNK
    mesh = plsc.VectorSubcoreMesh(core_axis_name="c", subcore_axis_name="s")

    @functools.partial(
        pl.kernel,
        out_type=jax.ShapeDtypeStruct((B, EMBED_DIM), jnp.float32),
        mesh=mesh,
        scratch_types=[
            pltpu.VMEM((b_per_w,), jnp.int32),
            pltpu.VMEM((NBUF, CHUNK, EMBED_DIM), jnp.float32),
            [pltpu.SemaphoreType.DMA] * NBUF,
            [pltpu.SemaphoreType.DMA] * NBUF,
        ],
    )
    def k(table_hbm, idx_hbm, out_hbm, idx_v, rows_v, gsems, osems):
        wid = lax.axis_index("s") * NUM_CORES + lax.axis_index("c")
        base = wid * b_per_w
        pltpu.sync_copy(idx_hbm.at[pl.ds(base, b_per_w)], idx_v)

        def issue_gather(j, buf):
            return pltpu.async_copy(
                table_hbm.at[idx_v.at[pl.ds(j * CHUNK, CHUNK)]],
                rows_v.at[buf],
                gsems[buf],
            )

        gathers = [None] * NBUF
        out_copies = [None] * NBUF
        gathers[0] = issue_gather(0, 0)
        for j in range(n_chunks):
            buf = j % NBUF
            if j + 1 < n_chunks:
                nb = (j + 1) % NBUF
                if out_copies[nb] is not None:
                    out_copies[nb].wait()
                gathers[nb] = issue_gather(j + 1, nb)
            gathers[buf].wait()
            out_copies[buf] = pltpu.async_copy(
                rows_v.at[buf],
                out_hbm.at[pl.ds(base + j * CHUNK, CHUNK)],
                osems[buf],
            )
        for c in out_copies:
            if c is not None:
                c.wait()

    return k(table, idx_flat)


def kernel(pos, A):
    batch, seq = pos.shape
    idx_flat = pos.reshape(batch * seq).astype(jnp.int32)
    out = _sc_gather(A, idx_flat)
    return out.reshape(batch, seq, EMBED_DIM)


# speedup vs baseline: 2.0800x; 1.0085x over previous
import functools

import jax
import jax.numpy as jnp
from jax import lax
from jax.experimental import pallas as pl
from jax.experimental.pallas import tpu as pltpu
from jax.experimental.pallas import tpu_sc as plsc

NUM_CORES = 2
NUM_SUBCORES = 16
NW = NUM_CORES * NUM_SUBCORES
CHUNK = 32
NBUF = 3


def _sc_gather(table, pos):
    batch, seq = pos.shape
    emb = table.shape[1]
    b_per_w = (batch * seq) // NW
    w_per_b = seq // b_per_w
    n_chunks = b_per_w // CHUNK
    mesh = plsc.VectorSubcoreMesh(core_axis_name="c", subcore_axis_name="s")

    @functools.partial(
        pl.kernel,
        out_type=jax.ShapeDtypeStruct((batch, seq, emb), jnp.float32),
        mesh=mesh,
        scratch_types=[
            pltpu.VMEM((b_per_w,), jnp.int32),
            pltpu.VMEM((NBUF, CHUNK, emb), jnp.float32),
            [pltpu.SemaphoreType.DMA] * NBUF,
            [pltpu.SemaphoreType.DMA] * NBUF,
        ],
    )
    def k(table_hbm, idx_hbm, out_hbm, idx_v, rows_v, gsems, osems):
        wid = lax.axis_index("s") * NUM_CORES + lax.axis_index("c")
        b = wid // w_per_b
        off = (wid % w_per_b) * b_per_w
        pltpu.sync_copy(idx_hbm.at[b, pl.ds(off, b_per_w)], idx_v)

        def issue_gather(j, buf):
            return pltpu.async_copy(
                table_hbm.at[idx_v.at[pl.ds(j * CHUNK, CHUNK)]],
                rows_v.at[buf],
                gsems[buf],
            )

        gathers = [None] * NBUF
        out_copies = [None] * NBUF
        gathers[0] = issue_gather(0, 0)
        for j in range(n_chunks):
            buf = j % NBUF
            if j + 1 < n_chunks:
                nb = (j + 1) % NBUF
                if out_copies[nb] is not None:
                    out_copies[nb].wait()
                gathers[nb] = issue_gather(j + 1, nb)
            gathers[buf].wait()
            out_copies[buf] = pltpu.async_copy(
                rows_v.at[buf],
                out_hbm.at[b, pl.ds(off + j * CHUNK, CHUNK)],
                osems[buf],
            )
        for c in out_copies:
            if c is not None:
                c.wait()

    return k(table, pos)


def kernel(pos, A):
    return _sc_gather(A, pos.astype(jnp.int32))
